# Initial kernel scaffold; baseline (speedup 1.0000x reference)
import jax
import jax.numpy as jnp
from jax.experimental import pallas as pl


def kernel(outputs, labels):
    raise NotImplementedError("write your pallas kernel here")


# speedup vs baseline: 6.9991x; 6.9991x over previous
import functools

import jax
import jax.numpy as jnp
from jax import lax
from jax.experimental import pallas as pl
from jax.experimental.pallas import tpu as pltpu

_R = 256


def _body(x_ref, lab_ref, out_ref, *, n, n_pad, nblocks, denom):
    i = pl.program_id(0)
    x = x_ref[...]
    lab = lab_ref[...]
    r = x.shape[0]

    xlo = x[:, :128]
    xhi = jnp.concatenate(
        [x[:, 128:], jnp.zeros((r, 256 - n), jnp.float32)], axis=1
    )
    labp = jnp.concatenate(
        [lab, jnp.zeros((r, n_pad - n), jnp.int32)], axis=1
    )

    chunks = []
    for h in range(n_pad // 128):
        idx = labp[:, h * 128 : (h + 1) * 128]
        in_lo = idx < 128
        a = jnp.take_along_axis(
            xlo, jnp.where(in_lo, idx, 0), axis=1, mode="promise_in_bounds"
        )
        b = jnp.take_along_axis(
            xhi, jnp.where(in_lo, 0, idx - 128), axis=1, mode="promise_in_bounds"
        )
        chunks.append(jnp.where(in_lo, a, b))
    g = jnp.concatenate(chunks, axis=1)

    valid = lax.broadcasted_iota(jnp.int32, (r, n_pad), 1) < n
    m = jnp.max(jnp.where(valid, g, -jnp.inf), axis=1, keepdims=True)
    e = jnp.where(valid, jnp.exp(g - m), 0.0)

    c = e
    shift = 1
    while shift < n_pad:
        c = c + jnp.concatenate(
            [jnp.zeros((r, shift), jnp.float32), c[:, : n_pad - shift]], axis=1
        )
        shift *= 2

    scores_sum = jnp.sum(jnp.where(valid, jnp.log(c), 0.0)) + n * jnp.sum(m)
    block_sum = scores_sum - jnp.sum(x)

    @pl.when(i == 0)
    def _():
        out_ref[0, 0] = 0.0

    out_ref[0, 0] += block_sum

    @pl.when(i == nblocks - 1)
    def _():
        out_ref[0, 0] = out_ref[0, 0] * denom


def kernel(outputs, labels):
    b, n = outputs.shape
    r = _R
    nblocks = b // r
    n_pad = 256
    body = functools.partial(
        _body, n=n, n_pad=n_pad, nblocks=nblocks, denom=1.0 / (b * n)
    )
    out = pl.pallas_call(
        body,
        grid=(nblocks,),
        in_specs=[
            pl.BlockSpec((r, n), lambda i: (i, 0)),
            pl.BlockSpec((r, n), lambda i: (i, 0)),
        ],
        out_specs=pl.BlockSpec(
            (1, 1), lambda i: (0, 0), memory_space=pltpu.SMEM
        ),
        out_shape=jax.ShapeDtypeStruct((1, 1), jnp.float32),
    )(outputs, labels)
    return out[0, 0]


# speedup vs baseline: 8.4162x; 1.2025x over previous
import functools

import jax
import jax.numpy as jnp
from jax import lax
from jax.experimental import pallas as pl
from jax.experimental.pallas import tpu as pltpu

_R = 512


def _body(x_ref, lab_ref, u_ref, out_ref, *, n, n_pad, nblocks, denom):
    i = pl.program_id(0)
    x = x_ref[...]
    lab = lab_ref[...]
    r = x.shape[0]

    xlo = x[:, :128]
    xhi = jnp.concatenate(
        [x[:, 128:], jnp.zeros((r, 256 - n), jnp.float32)], axis=1
    )
    labp = jnp.concatenate(
        [lab, jnp.zeros((r, n_pad - n), jnp.int32)], axis=1
    )

    chunks = []
    for h in range(n_pad // 128):
        idx = labp[:, h * 128 : (h + 1) * 128]
        in_lo = idx < 128
        a = jnp.take_along_axis(
            xlo, jnp.where(in_lo, idx, 0), axis=1, mode="promise_in_bounds"
        )
        b = jnp.take_along_axis(
            xhi, jnp.where(in_lo, 0, idx - 128), axis=1, mode="promise_in_bounds"
        )
        chunks.append(jnp.where(in_lo, a, b))
    g = jnp.concatenate(chunks, axis=1)

    valid = lax.broadcasted_iota(jnp.int32, (r, n_pad), 1) < n
    m = jnp.max(jnp.where(valid, g, -jnp.inf), axis=1, keepdims=True)
    e = jnp.where(valid, jnp.exp(g - m), 0.0)

    u = u_ref[...]
    e_hi = e.astype(jnp.bfloat16)
    e_lo = (e - e_hi.astype(jnp.float32)).astype(jnp.bfloat16)
    dims = (((1,), (0,)), ((), ()))
    c = lax.dot_general(
        e_hi, u, dims, preferred_element_type=jnp.float32
    ) + lax.dot_general(e_lo, u, dims, preferred_element_type=jnp.float32)

    scores_sum = jnp.sum(jnp.where(valid, jnp.log(c), 0.0)) + n * jnp.sum(m)
    block_sum = scores_sum - jnp.sum(x)

    @pl.when(i == 0)
    def _():
        out_ref[0, 0] = 0.0

    out_ref[0, 0] += block_sum

    @pl.when(i == nblocks - 1)
    def _():
        out_ref[0, 0] = out_ref[0, 0] * denom


def kernel(outputs, labels):
    b, n = outputs.shape
    r = _R
    nblocks = b // r
    n_pad = 256
    body = functools.partial(
        _body, n=n, n_pad=n_pad, nblocks=nblocks, denom=1.0 / (b * n)
    )
    u = (
        lax.broadcasted_iota(jnp.int32, (n_pad, n_pad), 0)
        <= lax.broadcasted_iota(jnp.int32, (n_pad, n_pad), 1)
    ).astype(jnp.bfloat16)
    out = pl.pallas_call(
        body,
        grid=(nblocks,),
        in_specs=[
            pl.BlockSpec((r, n), lambda i: (i, 0)),
            pl.BlockSpec((r, n), lambda i: (i, 0)),
            pl.BlockSpec((n_pad, n_pad), lambda i: (0, 0)),
        ],
        out_specs=pl.BlockSpec(
            (1, 1), lambda i: (0, 0), memory_space=pltpu.SMEM
        ),
        out_shape=jax.ShapeDtypeStruct((1, 1), jnp.float32),
    )(outputs, labels, u)
    return out[0, 0]
